# baseline (device time: 21278 ns/iter reference)
import jax
import jax.numpy as jnp
from jax import lax
from jax.experimental import pallas as pl
from jax.experimental.pallas import tpu as pltpu

B, Sq, Skv, Hq, Dh = 2, 128, 128, 8, 64
D = Hq * Dh
O_CH = Hq * Dh
L_OFF = O_CH
CH = O_CH + 2 * Hq
MASKS = (1, 3, 4)
N_STEP = len(MASKS)
BF = jnp.bfloat16


def _rep_heads(a):
    return jnp.broadcast_to(a[:, None, :], (Hq, Dh, Sq)).reshape(O_CH, Sq)


def _dot_t(a, b):
    return lax.dot_general(a, b, (((0,), (0,)), ((), ())),
                           preferred_element_type=jnp.float32)


def kernel(x, Wq, Wo, K_ext, V_ext):
    x2 = x.reshape(B * Sq, D)
    K2 = K_ext.reshape(B, Skv, D)
    V2 = V_ext.reshape(B, Skv, D)

    def body(x_ref, wq_ref, wo_ref, k_ref, v_ref, out_ref,
             acc, sbuf, rbuf, send_sems, recv_sems):
        my = lax.axis_index("i")

        rdmas = {}

        def issue(step, c):
            rdma = pltpu.make_async_remote_copy(
                src_ref=sbuf.at[step, c],
                dst_ref=rbuf.at[step, c],
                send_sem=send_sems.at[step, c],
                recv_sem=recv_sems.at[step, c],
                device_id=(my ^ MASKS[step],),
                device_id_type=pl.DeviceIdType.MESH,
            )
            rdma.start()
            rdmas[(step, c)] = rdma

        q = lax.dot_general(
            x_ref[...].astype(BF), wq_ref[...].astype(BF),
            (((1,), (0,)), ((), ())), preferred_element_type=jnp.float32)
        q_bf = (q * 0.125).astype(BF)
        wo_bf = wo_ref[...].astype(BF)

        barrier = pltpu.get_barrier_semaphore()
        for mask in MASKS:
            pl.semaphore_signal(
                barrier, inc=1,
                device_id=(my ^ mask,), device_id_type=pl.DeviceIdType.MESH,
            )
        pl.semaphore_wait(barrier, len(MASKS))

        for b in range(B):
            kb = k_ref[b].astype(BF)
            vb = v_ref[b].astype(BF)
            for h in range(Hq):
                qbh = q_bf[b * Sq:(b + 1) * Sq, h * Dh:(h + 1) * Dh]
                kbh = kb[:, h * Dh:(h + 1) * Dh]
                vbh = vb[:, h * Dh:(h + 1) * Dh]
                sT = lax.dot_general(
                    kbh, qbh, (((1,), (1,)), ((), ())),
                    preferred_element_type=jnp.float32)
                pT = jnp.exp(sT)
                lrow = jnp.sum(pT, axis=0, keepdims=True)
                oT = _dot_t(vbh, pT.astype(BF))
                acc[b, pl.ds(h * Dh, Dh), :] = oT
                acc[b, pl.ds(L_OFF + h, 1), :] = lrow
            acc[b, pl.ds(L_OFF + Hq, Hq), :] = jnp.zeros(
                (Hq, Sq), jnp.float32)
            sbuf[0, b] = acc[b].astype(BF)
            issue(0, b)

        for step in range(N_STEP):
            for c in range(B):
                rdmas[(step, c)].wait_recv()
                acc[c] = acc[c] + rbuf[step, c].astype(jnp.float32)
                if step + 1 < N_STEP:
                    sbuf[step + 1, c] = acc[c].astype(BF)
                    issue(step + 1, c)
                else:
                    linv = 1.0 / acc[c, L_OFF:L_OFF + Hq, :]
                    scaled = acc[c, 0:O_CH, :] * _rep_heads(linv)
                    out_ref[pl.ds(c * Sq, Sq), :] = _dot_t(
                        scaled.astype(BF), wo_bf)
        for rdma in rdmas.values():
            rdma.wait_send()

    out = pl.pallas_call(
        body,
        out_shape=jax.ShapeDtypeStruct((B * Sq, D), jnp.float32),
        in_specs=[pl.BlockSpec(memory_space=pltpu.VMEM)] * 5,
        out_specs=pl.BlockSpec(memory_space=pltpu.VMEM),
        scratch_shapes=[
            pltpu.VMEM((B, CH, Sq), jnp.float32),
            pltpu.VMEM((N_STEP, B, CH, Sq), jnp.bfloat16),
            pltpu.VMEM((N_STEP, B, CH, Sq), jnp.bfloat16),
            pltpu.SemaphoreType.DMA((N_STEP, B)),
            pltpu.SemaphoreType.DMA((N_STEP, B)),
        ],
        compiler_params=pltpu.CompilerParams(collective_id=0),
    )(x2, Wq, Wo, K2, V2)
    return out.reshape(B, Sq, D)


# device time: 18553 ns/iter; 1.1469x vs baseline; 1.1469x over previous
import jax
import jax.numpy as jnp
from jax import lax
from jax.experimental import pallas as pl
from jax.experimental.pallas import tpu as pltpu

B, Sq, Skv, Hq, Dh = 2, 128, 128, 8, 64
D = Hq * Dh
HG = 2
G = Hq // HG
NSC = B * G
O_SC = HG * Dh
SC_ROWS = 144
MASKS = (1, 3, 4)
ROTS = ((1, 3, 4), (3, 4, 1), (4, 1, 3))
N_STEP = 3
BF = jnp.bfloat16
F32 = jnp.float32


def _dot_t(a, b):
    return lax.dot_general(a, b, (((0,), (0,)), ((), ())),
                           preferred_element_type=F32)


def kernel(x, Wq, Wo, K_ext, V_ext):
    x2 = x.reshape(B * Sq, D).astype(BF)
    wq = Wq.astype(BF)
    wo = Wo.astype(BF)
    K2 = K_ext.reshape(B, Skv, D).astype(BF)
    V2 = V_ext.reshape(B, Skv, D).astype(BF)

    def body(x_ref, wq_ref, wo_ref, k_ref, v_ref, out_ref,
             sbuf, rbuf, send_sems, recv_sems, rsems):
        my = lax.axis_index("i")

        rdmas = {}

        def issue(step, sc):
            rdma = pltpu.make_async_remote_copy(
                src_ref=sbuf.at[step, sc],
                dst_ref=rbuf.at[step, sc],
                send_sem=send_sems.at[step, sc],
                recv_sem=recv_sems.at[step, sc],
                device_id=(my ^ ROTS[sc % 3][step],),
                device_id_type=pl.DeviceIdType.MESH,
            )
            rdma.start()
            rdmas[(step, sc)] = rdma

        q = lax.dot_general(
            x_ref[...], wq_ref[...],
            (((1,), (0,)), ((), ())), preferred_element_type=F32)
        q_bf = (q * 0.125).astype(BF)

        barrier = pltpu.get_barrier_semaphore()
        for j, mask in enumerate(MASKS):
            pl.semaphore_signal(
                barrier, inc=1,
                device_id=(my ^ mask,), device_id_type=pl.DeviceIdType.MESH,
            )
            pl.semaphore_signal(
                rsems.at[j], inc=1,
                device_id=(my ^ mask,), device_id_type=pl.DeviceIdType.MESH,
            )
        pl.semaphore_wait(barrier, len(MASKS))
        for j in range(len(MASKS)):
            pl.semaphore_wait(rsems.at[j], 1)

        zpad = jnp.zeros((SC_ROWS - O_SC - HG, Sq), BF)
        for b in range(B):
            kb = k_ref[b]
            vb = v_ref[b]
            for g in range(G):
                sc = b * G + g
                for hh in range(HG):
                    h = g * HG + hh
                    qbh = q_bf[b * Sq:(b + 1) * Sq, h * Dh:(h + 1) * Dh]
                    kbh = kb[:, h * Dh:(h + 1) * Dh]
                    vbh = vb[:, h * Dh:(h + 1) * Dh]
                    sT = lax.dot_general(
                        kbh, qbh, (((1,), (1,)), ((), ())),
                        preferred_element_type=F32)
                    pT = jnp.exp(sT)
                    lrow = jnp.sum(pT, axis=0, keepdims=True)
                    oT = _dot_t(vbh, pT.astype(BF))
                    sbuf[0, sc, pl.ds(hh * Dh, Dh), :] = oT.astype(BF)
                    sbuf[0, sc, pl.ds(O_SC + hh, 1), :] = lrow.astype(BF)
                sbuf[0, sc, pl.ds(O_SC + HG, SC_ROWS - O_SC - HG), :] = zpad
                issue(0, sc)

        parts = {}
        for step in range(N_STEP):
            for sc in range(NSC):
                rdmas[(step, sc)].wait_recv()
                if step + 1 < N_STEP:
                    sbuf[step + 1, sc] = sbuf[step, sc] + rbuf[step, sc]
                    issue(step + 1, sc)
                else:
                    blk = (sbuf[step, sc].astype(F32)
                           + rbuf[step, sc].astype(F32))
                    linv = 1.0 / blk[O_SC:O_SC + HG, :]
                    scale = jnp.broadcast_to(
                        linv[:, None, :], (HG, Dh, Sq)).reshape(O_SC, Sq)
                    scaled = (blk[0:O_SC, :] * scale).astype(BF)
                    g = sc % G
                    wo_slice = wo_ref[pl.ds(g * O_SC, O_SC), :]
                    part = _dot_t(scaled, wo_slice)
                    b = sc // G
                    parts.setdefault(b, []).append(part)
        for b in range(B):
            p = parts[b]
            out_ref[pl.ds(b * Sq, Sq), :] = (p[0] + p[1]) + (p[2] + p[3])

        for rdma in rdmas.values():
            rdma.wait_send()

    out = pl.pallas_call(
        body,
        out_shape=jax.ShapeDtypeStruct((B * Sq, D), F32),
        in_specs=[pl.BlockSpec(memory_space=pltpu.VMEM)] * 5,
        out_specs=pl.BlockSpec(memory_space=pltpu.VMEM),
        scratch_shapes=[
            pltpu.VMEM((N_STEP, NSC, SC_ROWS, Sq), BF),
            pltpu.VMEM((N_STEP, NSC, SC_ROWS, Sq), BF),
            pltpu.SemaphoreType.DMA((N_STEP, NSC)),
            pltpu.SemaphoreType.DMA((N_STEP, NSC)),
            pltpu.SemaphoreType.REGULAR((len(MASKS),)),
        ],
        compiler_params=pltpu.CompilerParams(collective_id=0),
    )(x2, wq, wo, K2, V2)
    return out.reshape(B, Sq, D)
